# baseline (speedup 1.0000x reference)
import functools

import jax
import jax.numpy as jnp
from jax import lax
from jax.experimental import pallas as pl
from jax.experimental.pallas import tpu as pltpu
from jax.experimental.pallas import tpu_sc as plsc

_IDX_CHUNK = 128


@functools.lru_cache(maxsize=None)
def _make_gather(B, V, D):
    info = plsc.get_sparse_core_info()
    NC, NS = info.num_cores, info.num_subcores
    NW = NC * NS
    assert B % (8 * NW) == 0
    b_per_w = B // NW
    chunk = min(_IDX_CHUNK, b_per_w)
    n_chunks = b_per_w // chunk
    assert b_per_w % chunk == 0

    mesh = plsc.VectorSubcoreMesh(core_axis_name="c", subcore_axis_name="s")

    @functools.partial(
        pl.kernel,
        mesh=mesh,
        out_type=jax.ShapeDtypeStruct((B, D), jnp.float32),
        compiler_params=pltpu.CompilerParams(use_tc_tiling_on_sc=False),
        scratch_types=[
            pltpu.VMEM((b_per_w,), jnp.int32),
            pltpu.VMEM((b_per_w, D), jnp.float32),
            pltpu.SemaphoreType.DMA,
            pltpu.SemaphoreType.DMA,
        ],
    )
    def gather_kernel(idx_hbm, table_hbm, out_hbm, idx_v, rows_v, gsem, osem):
        wid = lax.axis_index("s") * NC + lax.axis_index("c")
        base = wid * b_per_w
        pltpu.sync_copy(idx_hbm.at[pl.ds(base, b_per_w)], idx_v)
        for j in range(n_chunks):
            pltpu.async_copy(
                table_hbm.at[idx_v.at[pl.ds(j * chunk, chunk)]],
                rows_v.at[pl.ds(j * chunk, chunk)],
                gsem,
            )
        for j in range(n_chunks):
            pltpu.make_async_copy(
                table_hbm.at[idx_v.at[pl.ds(j * chunk, chunk)]],
                rows_v.at[pl.ds(j * chunk, chunk)],
                gsem,
            ).wait()
            pltpu.async_copy(
                rows_v.at[pl.ds(j * chunk, chunk)],
                out_hbm.at[pl.ds(base + j * chunk, chunk)],
                osem,
            )
        for j in range(n_chunks):
            pltpu.make_async_copy(
                rows_v.at[pl.ds(j * chunk, chunk)],
                out_hbm.at[pl.ds(base + j * chunk, chunk)],
                osem,
            ).wait()

    return gather_kernel


def kernel(u_idx, user_emb):
    (B,) = u_idx.shape
    V, D = user_emb.shape
    return _make_gather(B, V, D)(u_idx, user_emb)


# speedup vs baseline: 3.8863x; 3.8863x over previous
import functools

import jax
import jax.numpy as jnp
from jax import lax
from jax.experimental import pallas as pl
from jax.experimental.pallas import tpu as pltpu
from jax.experimental.pallas import tpu_sc as plsc

L = 16


@functools.lru_cache(maxsize=None)
def _make_gather(B, V, D):
    assert (B, V, D) == (16384, 1000000, 64)
    NC, NS = 2, 16

    CHUNK = 512
    TAIL_LO = 999936

    mesh = plsc.VectorSubcoreMesh(
        core_axis_name="c", subcore_axis_name="s", num_cores=NC, num_subcores=NS
    )

    @functools.partial(
        pl.kernel,
        mesh=mesh,
        out_type=jax.ShapeDtypeStruct((B * D,), jnp.float32),
        compiler_params=pltpu.CompilerParams(
            use_tc_tiling_on_sc=True, needs_layout_passes=False
        ),
        scratch_types=[
            pltpu.VMEM((B,), jnp.int32),
            pltpu.VMEM((B + L,), jnp.int32),
            pltpu.VMEM((B + L,), jnp.int32),
            pltpu.VMEM((D, CHUNK), jnp.float32),
            pltpu.VMEM((D, CHUNK), jnp.float32),
            pltpu.VMEM((64, 128), jnp.float32),
            pltpu.VMEM((L * D,), jnp.float32),
            pltpu.SemaphoreType.DMA,
            pltpu.SemaphoreType.DMA,
            pltpu.SemaphoreType.DMA,
        ],
    )
    def gather_kernel(
        idx_hbm, table_hbm, tail_hbm, out_hbm,
        uall_v, lp_v, cl_v, bufA, bufB, tail_v, rowring,
        semA, semB, osem,
    ):
        w = lax.axis_index("s") * NC + lax.axis_index("c")
        lo_col = 244 * w + jnp.minimum(w, 4)
        ulo = lo_col * 128
        uhi = ulo + 244 * 128 + jnp.where(w < 4, 128, 0) + jnp.where(w == 31, 64, 0)

        iota = lax.iota(jnp.int32, L)

        pltpu.sync_copy(idx_hbm, uall_v)

        def scan_body(k, n_vec):
            v = uall_v[pl.ds(pl.multiple_of(k * L, L), L)]
            mask = (v >= ulo) & (v < uhi)
            mi = mask.astype(jnp.int32)
            excl = plsc.cumsum(mi) - mi
            plsc.store_scatter(lp_v, [n_vec + excl], k * L + iota, mask=mask)
            return n_vec + plsc.all_reduce_population_count(mask)

        n_vec = lax.fori_loop(0, B // L, scan_body, jnp.zeros((L,), jnp.int32))
        cnt = jnp.max(n_vec)
        nk = (cnt + L - 1) // L

        def process(buf, clo, cwidth, gcnt):
            chi = clo + cwidth

            def rescan_body(k2, m_vec):
                p = lp_v[pl.ds(pl.multiple_of(k2 * L, L), L)] & (B - 1)
                lane_ok = (k2 * L + iota) < cnt
                u = plsc.load_gather(uall_v, [p], mask=lane_ok)
                mask = lane_ok & (u >= clo) & (u < chi)
                word = (u - clo) | (p << 9)
                mi = mask.astype(jnp.int32)
                excl = plsc.cumsum(mi) - mi
                plsc.store_scatter(cl_v, [m_vec + excl], word, mask=mask)
                return m_vec + plsc.all_reduce_population_count(mask)

            m_vec = lax.fori_loop(0, nk, rescan_body, jnp.zeros((L,), jnp.int32))
            m = jnp.max(m_vec)

            def group_body(j, gcnt):
                words = cl_v[pl.ds(pl.multiple_of(j * L, L), L)]
                l_vec = words & (buf.shape[1] - 1)
                lane_valid = (j * L + iota) < m
                for d in range(D):
                    vals = plsc.load_gather(
                        buf, [jnp.full((L,), d, jnp.int32), l_vec], mask=lane_valid
                    )
                    plsc.store_scatter(rowring, [iota * D + d], vals, mask=lane_valid)
                for q in range(L):
                    valid = (j * L + q) < m
                    pos = words[q] >> 9

                    @pl.when(valid)
                    def _():
                        pltpu.async_copy(
                            rowring.at[pl.ds(q * D, D)],
                            out_hbm.at[pl.ds(pl.multiple_of(pos * D, 8), D)],
                            osem,
                        )

                for q in range(L):
                    valid = (j * L + q) < m

                    @pl.when(valid)
                    def _():
                        pltpu.make_async_copy(
                            rowring.at[pl.ds(q * D, D)],
                            out_hbm.at[pl.ds(0, D)],
                            osem,
                        ).wait()

                return gcnt

            return lax.fori_loop(0, (m + L - 1) // L, group_body, gcnt)

        def fire(c_idx, buf, sem):
            coff = pl.multiple_of((lo_col + 4 * c_idx) * 128, 128)
            pltpu.async_copy(table_hbm.at[:, pl.ds(coff, CHUNK)], buf, sem)

        def wait(c_idx, buf, sem):
            coff = pl.multiple_of((lo_col + 4 * c_idx) * 128, 128)
            pltpu.make_async_copy(
                table_hbm.at[:, pl.ds(coff, CHUNK)], buf, sem
            ).wait()

        fire(0, bufA, semA)
        fire(1, bufB, semB)

        def chunk_body(k, gcnt):
            ca = 2 * k
            cb = 2 * k + 1
            wait(ca, bufA, semA)
            gcnt = process(bufA, ulo + ca * CHUNK, CHUNK, gcnt)
            fire(jnp.minimum(ca + 2, 60), bufA, semA)
            wait(cb, bufB, semB)
            gcnt = process(bufB, ulo + cb * CHUNK, CHUNK, gcnt)
            fire(jnp.minimum(cb + 2, 60), bufB, semB)
            return gcnt

        gcnt = lax.fori_loop(0, 30, chunk_body, jnp.zeros((), jnp.int32))
        wait(60, bufA, semA)
        gcnt = process(bufA, ulo + 60 * CHUNK, CHUNK, gcnt)
        wait(60, bufB, semB)

        extra_off = pl.multiple_of((lo_col + 244) * 128, 128)
        extra_lo = ulo + 61 * CHUNK

        @pl.when(w != 31)
        def _():
            pltpu.async_copy(
                table_hbm.at[:, pl.ds(extra_off, 128)], bufA.at[:, pl.ds(0, 128)], semA
            )
            pltpu.make_async_copy(
                table_hbm.at[:, pl.ds(extra_off, 128)], bufA.at[:, pl.ds(0, 128)], semA
            ).wait()

        gcnt = process(bufA, extra_lo, jnp.where(w < 4, 128, 0), gcnt)

        pltpu.sync_copy(tail_hbm, tail_v)
        gcnt = process(tail_v, TAIL_LO, jnp.where(w == 31, 64, 0), gcnt)

    return gather_kernel


def kernel(u_idx, user_emb):
    (B,) = u_idx.shape
    V, D = user_emb.shape
    tail = jnp.pad(user_emb[999936:, :].T, ((0, 0), (0, 64)))
    flat = _make_gather(B, V, D)(u_idx, user_emb.T, tail)
    return flat.reshape(B, D)


# speedup vs baseline: 3.9041x; 1.0046x over previous
import functools

import jax
import jax.numpy as jnp
from jax import lax
from jax.experimental import pallas as pl
from jax.experimental.pallas import tpu as pltpu
from jax.experimental.pallas import tpu_sc as plsc

L = 16


@functools.lru_cache(maxsize=None)
def _make_gather(B, V, D):
    assert (B, V, D) == (16384, 1000000, 64)
    NC, NS = 2, 16

    CHUNK = 512
    TAIL_LO = 999936

    mesh = plsc.VectorSubcoreMesh(
        core_axis_name="c", subcore_axis_name="s", num_cores=NC, num_subcores=NS
    )

    @functools.partial(
        pl.kernel,
        mesh=mesh,
        out_type=jax.ShapeDtypeStruct((B * D,), jnp.float32),
        compiler_params=pltpu.CompilerParams(
            use_tc_tiling_on_sc=True, needs_layout_passes=False
        ),
        scratch_types=[
            pltpu.VMEM((B,), jnp.int32),
            pltpu.VMEM((B + L,), jnp.int32),
            pltpu.VMEM((B + L,), jnp.int32),
            pltpu.VMEM((D, CHUNK), jnp.float32),
            pltpu.VMEM((D, CHUNK), jnp.float32),
            pltpu.VMEM((64, 128), jnp.float32),
            pltpu.VMEM((2 * L * D,), jnp.float32),
            pltpu.SemaphoreType.DMA,
            pltpu.SemaphoreType.DMA,
            pltpu.SemaphoreType.DMA,
        ],
    )
    def gather_kernel(
        idx_hbm, table_hbm, tail_hbm, out_hbm,
        uall_v, lp_v, cl_v, bufA, bufB, tail_v, rowring,
        semA, semB, osem,
    ):
        w = lax.axis_index("s") * NC + lax.axis_index("c")
        lo_col = 244 * w + jnp.minimum(w, 4)
        ulo = lo_col * 128
        uhi = ulo + 244 * 128 + jnp.where(w < 4, 128, 0) + jnp.where(w == 31, 64, 0)

        iota = lax.iota(jnp.int32, L)

        fire0_coff = pl.multiple_of(lo_col * 128, 128)
        pltpu.async_copy(table_hbm.at[:, pl.ds(fire0_coff, CHUNK)], bufA, semA)
        fire1_coff = pl.multiple_of((lo_col + 4) * 128, 128)
        pltpu.async_copy(table_hbm.at[:, pl.ds(fire1_coff, CHUNK)], bufB, semB)

        pltpu.sync_copy(idx_hbm, uall_v)

        def scan_body(k, n_vec):
            v = uall_v[pl.ds(pl.multiple_of(k * L, L), L)]
            mask = (v >= ulo) & (v < uhi)
            mi = mask.astype(jnp.int32)
            excl = plsc.cumsum(mi) - mi
            plsc.store_scatter(lp_v, [n_vec + excl], k * L + iota, mask=mask)
            return n_vec + plsc.all_reduce_population_count(mask)

        n_vec = lax.fori_loop(0, B // L, scan_body, jnp.zeros((L,), jnp.int32))
        cnt = jnp.max(n_vec)
        nk = (cnt + L - 1) // L

        def process(buf, clo, cwidth, gcnt):
            chi = clo + cwidth

            def rescan_body(k2, m_vec):
                p = lp_v[pl.ds(pl.multiple_of(k2 * L, L), L)] & (B - 1)
                lane_ok = (k2 * L + iota) < cnt
                u = plsc.load_gather(uall_v, [p], mask=lane_ok)
                mask = lane_ok & (u >= clo) & (u < chi)
                word = (u - clo) | (p << 9)
                mi = mask.astype(jnp.int32)
                excl = plsc.cumsum(mi) - mi
                plsc.store_scatter(cl_v, [m_vec + excl], word, mask=mask)
                return m_vec + plsc.all_reduce_population_count(mask)

            m_vec = lax.fori_loop(0, nk, rescan_body, jnp.zeros((L,), jnp.int32))
            m = jnp.max(m_vec)

            def group_body(j, carry):
                gcnt, p0, p1 = carry
                parity = gcnt & 1
                base = pl.multiple_of(parity * (L * D), D)
                pend = jnp.where(parity == 0, p0, p1)

                def drain_one(_, x):
                    pltpu.make_async_copy(
                        rowring.at[pl.ds(0, D)], out_hbm.at[pl.ds(0, D)], osem
                    ).wait()
                    return x

                lax.fori_loop(0, pend, drain_one, 0)

                words = cl_v[pl.ds(pl.multiple_of(j * L, L), L)]
                l_vec = words & (buf.shape[1] - 1)
                lane_valid = (j * L + iota) < m
                for d in range(D):
                    vals = plsc.load_gather(
                        buf, [jnp.full((L,), d, jnp.int32), l_vec], mask=lane_valid
                    )
                    plsc.store_scatter(
                        rowring, [base + iota * D + d], vals, mask=lane_valid
                    )
                for q in range(L):
                    valid = (j * L + q) < m
                    pos = words[q] >> 9

                    @pl.when(valid)
                    def _():
                        pltpu.async_copy(
                            rowring.at[pl.ds(base + q * D, D)],
                            out_hbm.at[pl.ds(pl.multiple_of(pos * D, 8), D)],
                            osem,
                        )

                nvalid = jnp.minimum(L, m - j * L)
                p0 = jnp.where(parity == 0, nvalid, p0)
                p1 = jnp.where(parity == 1, nvalid, p1)
                return (gcnt + 1, p0, p1)

            return lax.fori_loop(0, (m + L - 1) // L, group_body, gcnt)

        def fire(c_idx, buf, sem):
            coff = pl.multiple_of((lo_col + 4 * c_idx) * 128, 128)
            pltpu.async_copy(table_hbm.at[:, pl.ds(coff, CHUNK)], buf, sem)

        def wait(c_idx, buf, sem):
            coff = pl.multiple_of((lo_col + 4 * c_idx) * 128, 128)
            pltpu.make_async_copy(
                table_hbm.at[:, pl.ds(coff, CHUNK)], buf, sem
            ).wait()

        def chunk_body(k, gcnt):
            ca = 2 * k
            cb = 2 * k + 1
            wait(ca, bufA, semA)
            gcnt = process(bufA, ulo + ca * CHUNK, CHUNK, gcnt)
            fire(jnp.minimum(ca + 2, 60), bufA, semA)
            wait(cb, bufB, semB)
            gcnt = process(bufB, ulo + cb * CHUNK, CHUNK, gcnt)
            fire(jnp.minimum(cb + 2, 60), bufB, semB)
            return gcnt

        gcnt = lax.fori_loop(0, 30, chunk_body, (jnp.zeros((), jnp.int32),) * 3)
        wait(60, bufA, semA)
        gcnt = process(bufA, ulo + 60 * CHUNK, CHUNK, gcnt)
        wait(60, bufB, semB)

        extra_off = pl.multiple_of((lo_col + 244) * 128, 128)
        extra_lo = ulo + 61 * CHUNK

        @pl.when(w != 31)
        def _():
            pltpu.async_copy(
                table_hbm.at[:, pl.ds(extra_off, 128)], bufA.at[:, pl.ds(0, 128)], semA
            )
            pltpu.make_async_copy(
                table_hbm.at[:, pl.ds(extra_off, 128)], bufA.at[:, pl.ds(0, 128)], semA
            ).wait()

        gcnt = process(bufA, extra_lo, jnp.where(w < 4, 128, 0), gcnt)

        pltpu.sync_copy(tail_hbm, tail_v)
        gcnt = process(tail_v, TAIL_LO, jnp.where(w == 31, 64, 0), gcnt)

        def final_drain(_, x):
            pltpu.make_async_copy(
                rowring.at[pl.ds(0, D)], out_hbm.at[pl.ds(0, D)], osem
            ).wait()
            return x

        lax.fori_loop(0, gcnt[1] + gcnt[2], final_drain, 0)

    return gather_kernel


def kernel(u_idx, user_emb):
    (B,) = u_idx.shape
    V, D = user_emb.shape
    tail = jnp.pad(user_emb[999936:, :].T, ((0, 0), (0, 64)))
    flat = _make_gather(B, V, D)(u_idx, user_emb.T, tail)
    return flat.reshape(B, D)
